# baseline (speedup 1.0000x reference)
import functools

import jax
import jax.numpy as jnp
from jax import lax
from jax.experimental import pallas as pl
from jax.experimental.pallas import tpu as pltpu
from jax.experimental.pallas import tpu_sc as plsc

NC = 2
NS = 16
NW = NC * NS
LANES = 16
CHUNK = 128


def _make_sc_gather(B, S, D, FL, mask_row):
  out_len = FL + 1
  b_per_w = B // NW
  n_chunks = FL // CHUNK
  mesh = plsc.VectorSubcoreMesh(core_axis_name="c", subcore_axis_name="s")

  @functools.partial(
      pl.kernel,
      mesh=mesh,
      out_type=jax.ShapeDtypeStruct((B * out_len, D), jnp.float32),
      compiler_params=pltpu.CompilerParams(use_tc_tiling_on_sc=False),
      scratch_types=[
          pltpu.VMEM((CHUNK,), jnp.int32),
          pltpu.VMEM((CHUNK,), jnp.int32),
          pltpu.VMEM((CHUNK, D), jnp.float32),
          pltpu.VMEM((1, D), jnp.float32),
          pltpu.SemaphoreType.DMA,
      ],
  )
  def k(table_hbm, idx_hbm, out_hbm, idx_v, g_v, rows_v, gt_v, sem):
    wid = lax.axis_index("s") * NC + lax.axis_index("c")
    for i in range(b_per_w):
      b = wid * b_per_w + i
      pltpu.sync_copy(table_hbm.at[pl.ds(b * S, 1)], gt_v)
      pltpu.sync_copy(gt_v, out_hbm.at[pl.ds(b * out_len, 1)])
      for c in range(n_chunks):
        pltpu.sync_copy(idx_hbm.at[pl.ds(b * FL + c * CHUNK, CHUNK)], idx_v)
        for gidx in range(CHUNK // LANES):
          v = idx_v[pl.ds(gidx * LANES, LANES)]
          valid = v < (S - 1)
          g = jnp.where(valid, v + (b * S + 1), mask_row)
          g_v[pl.ds(gidx * LANES, LANES)] = g
        pltpu.async_copy(table_hbm.at[g_v], rows_v, sem).wait()
        pltpu.sync_copy(rows_v, out_hbm.at[pl.ds(b * out_len + 1 + c * CHUNK, CHUNK)])

  return k


@jax.jit
def kernel(data, revert_idx, mask_token):
  B, S, D = data.shape
  FL = revert_idx.shape[-1]
  data_flat = data.reshape(B * S, D)
  mask_rows = jnp.broadcast_to(mask_token[None, :], (8, D))
  table = jnp.concatenate([data_flat, mask_rows], axis=0)
  idx_flat = revert_idx.reshape(-1)
  sc = _make_sc_gather(B, S, D, FL, B * S)
  out_flat = sc(table, idx_flat)
  return out_flat.reshape(B, FL + 1, D)


# speedup vs baseline: 1.0002x; 1.0002x over previous
import functools

import jax
import jax.numpy as jnp
from jax import lax
from jax.experimental import pallas as pl
from jax.experimental.pallas import tpu as pltpu
from jax.experimental.pallas import tpu_sc as plsc

NC = 2
NS = 16
NW = NC * NS
LANES = 16
CHUNK = 128
NBUF = 3


def _make_sc_gather(B, S, D, FL, mask_row):
  out_len = FL + 1
  b_per_w = B // NW
  rows_per_w = b_per_w * FL
  n_chunks = rows_per_w // CHUNK
  mesh = plsc.VectorSubcoreMesh(core_axis_name="c", subcore_axis_name="s")

  @functools.partial(
      pl.kernel,
      mesh=mesh,
      out_type=jax.ShapeDtypeStruct((B * out_len, D), jnp.float32),
      scratch_types=[
          pltpu.VMEM((rows_per_w,), jnp.int32),
          pltpu.VMEM((n_chunks, CHUNK), jnp.int32),
          pltpu.VMEM((n_chunks, CHUNK), jnp.int32),
          pltpu.VMEM((NBUF, CHUNK, D), jnp.float32),
          pltpu.VMEM((1, B), jnp.int32),
          pltpu.VMEM((1, B), jnp.int32),
          pltpu.VMEM((B, D), jnp.float32),
          pltpu.SemaphoreType.DMA,
          pltpu.SemaphoreType.DMA,
          pltpu.SemaphoreType.DMA,
      ],
  )
  def k(table_hbm, idx_hbm, out_hbm, idx_v, g_v, pos_v, rows_v, gsrc_v,
        gdst_v, grows_v, gsem, ssem, g2sem):
    wid = lax.axis_index("s") * NC + lax.axis_index("c")
    iota = lax.iota(jnp.int32, LANES)

    pltpu.sync_copy(idx_hbm.at[pl.ds(wid * rows_per_w, rows_per_w)], idx_v)

    for c in range(n_chunks):
      b = wid * b_per_w + c // (FL // CHUNK)
      cc = c % (FL // CHUNK)
      tbase = b * S + 1
      obase = b * out_len + 1 + cc * CHUNK
      for gg in range(CHUNK // LANES):
        v = idx_v[pl.ds(c * CHUNK + gg * LANES, LANES)]
        g = jnp.where(v < (S - 1), v + tbase, mask_row)
        g_v[c, pl.ds(gg * LANES, LANES)] = g
        pos_v[c, pl.ds(gg * LANES, LANES)] = iota + (obase + gg * LANES)

    gd = [None] * n_chunks
    sd = [None] * n_chunks
    for c in range(n_chunks):
      if c >= NBUF:
        sd[c - NBUF].wait()
      gd[c] = pltpu.async_copy(table_hbm.at[g_v.at[c]], rows_v.at[c % NBUF],
                               gsem)
      if c >= 1:
        gd[c - 1].wait()
        sd[c - 1] = pltpu.async_copy(rows_v.at[(c - 1) % NBUF],
                                     out_hbm.at[pos_v.at[c - 1]], ssem)
    gd[n_chunks - 1].wait()
    sd[n_chunks - 1] = pltpu.async_copy(rows_v.at[(n_chunks - 1) % NBUF],
                                        out_hbm.at[pos_v.at[n_chunks - 1]],
                                        ssem)
    for c in range(n_chunks - NBUF, n_chunks):
      sd[c].wait()

    @pl.when(wid == 0)
    def _():
      for gg in range(B // LANES):
        lane = iota + gg * LANES
        gsrc_v[0, pl.ds(gg * LANES, LANES)] = lane * S
        gdst_v[0, pl.ds(gg * LANES, LANES)] = lane * out_len
      pltpu.async_copy(table_hbm.at[gsrc_v.at[0]], grows_v, g2sem).wait()
      pltpu.async_copy(grows_v, out_hbm.at[gdst_v.at[0]], g2sem).wait()

  return k


@jax.jit
def kernel(data, revert_idx, mask_token):
  B, S, D = data.shape
  FL = revert_idx.shape[-1]
  data_flat = data.reshape(B * S, D)
  mask_rows = jnp.broadcast_to(mask_token[None, :], (8, D))
  table = jnp.concatenate([data_flat, mask_rows], axis=0)
  idx_flat = revert_idx.reshape(-1)
  sc = _make_sc_gather(B, S, D, FL, B * S)
  out_flat = sc(table, idx_flat)
  return out_flat.reshape(B, FL + 1, D)


# speedup vs baseline: 13.0799x; 13.0770x over previous
import functools

import jax
import jax.numpy as jnp
from jax import lax
from jax.experimental import pallas as pl
from jax.experimental.pallas import tpu as pltpu
from jax.experimental.pallas import tpu_sc as plsc

NC = 2
NS = 16
NW = NC * NS
LANES = 16
CHUNK = 128
NBUF = 3


def _make_sc_gather(B, S, D, FL, mask_row):
  out_len = FL + 1
  b_per_w = B // NW
  rows_per_w = b_per_w * FL
  n_chunks = rows_per_w // CHUNK
  mesh = plsc.VectorSubcoreMesh(core_axis_name="c", subcore_axis_name="s")

  @functools.partial(
      pl.kernel,
      mesh=mesh,
      out_type=jax.ShapeDtypeStruct((B * out_len, D), jnp.float32),
      scratch_types=[
          pltpu.VMEM((rows_per_w,), jnp.int32),
          pltpu.VMEM((n_chunks, CHUNK), jnp.int32),
          pltpu.VMEM((n_chunks, CHUNK), jnp.int32),
          pltpu.VMEM((NBUF, CHUNK, D), jnp.float32),
          pltpu.VMEM((1, B), jnp.int32),
          pltpu.VMEM((1, B), jnp.int32),
          pltpu.VMEM((B, D), jnp.float32),
          pltpu.SemaphoreType.DMA,
          pltpu.SemaphoreType.DMA,
          pltpu.SemaphoreType.DMA,
      ],
  )
  def k(table_hbm, idx_hbm, out_hbm, idx_v, g_v, pos_v, rows_v, gsrc_v,
        gdst_v, grows_v, gsem, ssem, g2sem):
    wid = lax.axis_index("s") * NC + lax.axis_index("c")
    iota = lax.iota(jnp.int32, LANES)

    pltpu.sync_copy(idx_hbm.at[pl.ds(wid * rows_per_w, rows_per_w)], idx_v)

    for c in range(n_chunks):
      b = wid * b_per_w + c // (FL // CHUNK)
      cc = c % (FL // CHUNK)
      tbase = b * S + 1
      obase = b * out_len + 1 + cc * CHUNK
      for gg in range(CHUNK // LANES):
        v = idx_v[pl.ds(c * CHUNK + gg * LANES, LANES)]
        g = jnp.where(v < (S - 1), v + tbase, v + (mask_row - (S - 1)))
        g_v[c, pl.ds(gg * LANES, LANES)] = g
        pos_v[c, pl.ds(gg * LANES, LANES)] = iota + (obase + gg * LANES)

    gd = [None] * n_chunks
    sd = [None] * n_chunks
    for c in range(n_chunks):
      if c >= NBUF:
        sd[c - NBUF].wait()
      gd[c] = pltpu.async_copy(table_hbm.at[g_v.at[c]], rows_v.at[c % NBUF],
                               gsem)
      if c >= 1:
        gd[c - 1].wait()
        sd[c - 1] = pltpu.async_copy(rows_v.at[(c - 1) % NBUF],
                                     out_hbm.at[pos_v.at[c - 1]], ssem)
    gd[n_chunks - 1].wait()
    sd[n_chunks - 1] = pltpu.async_copy(rows_v.at[(n_chunks - 1) % NBUF],
                                        out_hbm.at[pos_v.at[n_chunks - 1]],
                                        ssem)
    for c in range(n_chunks - NBUF, n_chunks):
      sd[c].wait()

    @pl.when(wid == 0)
    def _():
      for gg in range(B // LANES):
        lane = iota + gg * LANES
        gsrc_v[0, pl.ds(gg * LANES, LANES)] = lane * S
        gdst_v[0, pl.ds(gg * LANES, LANES)] = lane * out_len
      pltpu.async_copy(table_hbm.at[gsrc_v.at[0]], grows_v, g2sem).wait()
      pltpu.async_copy(grows_v, out_hbm.at[gdst_v.at[0]], g2sem).wait()

  return k


@jax.jit
def kernel(data, revert_idx, mask_token):
  B, S, D = data.shape
  FL = revert_idx.shape[-1]
  data_flat = data.reshape(B * S, D)
  mask_rows = jnp.broadcast_to(mask_token[None, :], (FL - (S - 1), D))
  table = jnp.concatenate([data_flat, mask_rows], axis=0)
  idx_flat = revert_idx.reshape(-1)
  sc = _make_sc_gather(B, S, D, FL, B * S)
  out_flat = sc(table, idx_flat)
  return out_flat.reshape(B, FL + 1, D)
